# baseline (speedup 1.0000x reference)
import dataclasses
import functools

import jax
import jax.numpy as jnp
from jax import lax
from jax.experimental import pallas as pl
from jax.experimental.pallas import tpu as pltpu
from jax.experimental.pallas import tpu_sc as plsc

N = 10000
E = 320000
D = 768
L = 4
H = 128
NC = 2
NS = 16
NW = NC * NS
GW = 96
GH = GW // 2
GROUPS = 108
GA = GROUPS + 3
EP = NW * GA * GW
NP = 10240
RPT = NP // NS
DPT = NP // NS

BLK = 1000
GRID = N // BLK

_mesh = plsc.VectorSubcoreMesh(core_axis_name="core", subcore_axis_name="subcore")

_sc_params = pltpu.CompilerParams()
if "needs_layout_passes" in pltpu.CompilerParams.__dataclass_fields__:
    _sc_params = dataclasses.replace(_sc_params, needs_layout_passes=False)


@functools.partial(
    pl.kernel,
    out_type=jax.ShapeDtypeStruct((NC, 1, NP), jnp.float32),
    mesh=_mesh,
    scratch_types=[
        pltpu.VMEM((GA, GW), jnp.int32),
        pltpu.VMEM((GA, GW), jnp.float32),
        pltpu.VMEM_SHARED((NP,), jnp.float32),
    ],
)
def _sc_degree(dstg_hbm, ewg_hbm, zeros_hbm, out_hbm, dst_v, ew_v, acc_sh):
    c = lax.axis_index("core")
    s = lax.axis_index("subcore")
    wid = s * NC + c
    pltpu.sync_copy(zeros_hbm, acc_sh.at[pl.ds(s * DPT, DPT)])
    plsc.subcore_barrier()
    pltpu.sync_copy(dstg_hbm.at[wid], dst_v)
    pltpu.sync_copy(ewg_hbm.at[wid], ew_v)

    @pl.loop(0, GA)
    def _(g):
        pltpu.sync_copy(ew_v.at[g], acc_sh.at[dst_v.at[g]], add=True)

    plsc.subcore_barrier()
    pltpu.sync_copy(acc_sh.at[pl.ds(s * DPT, DPT)],
                    out_hbm.at[c, 0, pl.ds(s * DPT, DPT)])


@functools.partial(
    pl.kernel,
    out_type=jax.ShapeDtypeStruct((NC, NP, H), jnp.float32),
    mesh=_mesh,
    scratch_types=[
        pltpu.VMEM((3, 128), jnp.int32),
        pltpu.VMEM((3, 128), jnp.int32),
        pltpu.VMEM((3, 128), jnp.int32),
        pltpu.VMEM((2, GH), jnp.int32),
        pltpu.VMEM((GW, H), jnp.float32),
        pltpu.VMEM((GW, H), jnp.float32),
        pltpu.VMEM((GW, H), jnp.float32),
        pltpu.VMEM((GH, H), jnp.float32),
        pltpu.VMEM_SHARED((NP, H), jnp.float32),
        pltpu.SemaphoreType.DMA,
        pltpu.SemaphoreType.DMA,
        pltpu.SemaphoreType.DMA,
        pltpu.SemaphoreType.DMA,
        pltpu.SemaphoreType.DMA,
        pltpu.SemaphoreType.DMA,
    ],
    compiler_params=_sc_params,
)
def _sc_message(hs_hbm, packed_hbm, zrows_hbm, out_hbm,
                idx_a, idx_b, idx_c, dst2, rows_a, rows_b, rows_c,
                rs, acc_sh, sem_ia, sem_ib, sem_ic, sem_a, sem_b, sem_c):
    c = lax.axis_index("core")
    s = lax.axis_index("subcore")
    wid = s * NC + c
    pltpu.sync_copy(zrows_hbm, acc_sh.at[pl.ds(s * RPT, RPT)])
    plsc.subcore_barrier()

    two = lax.broadcast(2, (16,))

    def idx_dma(g, idx, sem):
        pltpu.async_copy(packed_hbm.at[wid, g], idx, sem)

    def idx_wait(g, idx, sem):
        pltpu.make_async_copy(packed_hbm.at[wid, g], idx, sem).wait()

    def gather(idx, rows, sem):
        pltpu.async_copy(hs_hbm.at[idx.at[0, pl.ds(0, GW)]], rows, sem)

    def gather_wait(idx, rows, sem):
        pltpu.make_async_copy(hs_hbm.at[idx.at[0, pl.ds(0, GW)]],
                              rows, sem).wait()

    def process(idx, rows, sem):
        gather_wait(idx, rows, sem)
        for t in range(2):
            for k in range(GH // 16):
                dst2[t, pl.ds(k * 16, 16)] = \
                    idx[1, pl.ds(t * GH + k * 16, 16)]
        for t in range(2):
            base = t * GH

            @pl.loop(0, GH // 2)
            def _(pp):
                for u in range(2):
                    e = pp * 2 + u
                    e2 = e + base
                    wvi = plsc.load_gather(
                        idx, [two, lax.broadcast(e2, (16,))])
                    wv = plsc.bitcast(wvi, jnp.float32)
                    for k in range(H // 16):
                        rs[e, pl.ds(k * 16, 16)] = \
                            rows[e2, pl.ds(k * 16, 16)] * wv

            pltpu.sync_copy(rs, acc_sh.at[dst2.at[t]], add=True)

    idx_dma(0, idx_a, sem_ia)
    idx_dma(1, idx_b, sem_ib)
    idx_wait(0, idx_a, sem_ia)
    gather(idx_a, rows_a, sem_a)
    idx_wait(1, idx_b, sem_ib)
    gather(idx_b, rows_b, sem_b)
    idx_dma(2, idx_c, sem_ic)

    @pl.loop(0, GROUPS, step=3)
    def _(g):
        idx_wait(g + 2, idx_c, sem_ic)
        gather(idx_c, rows_c, sem_c)
        process(idx_a, rows_a, sem_a)
        idx_dma(g + 3, idx_a, sem_ia)
        idx_wait(g + 3, idx_a, sem_ia)
        gather(idx_a, rows_a, sem_a)
        process(idx_b, rows_b, sem_b)
        idx_dma(g + 4, idx_b, sem_ib)
        idx_wait(g + 4, idx_b, sem_ib)
        gather(idx_b, rows_b, sem_b)
        process(idx_c, rows_c, sem_c)
        idx_dma(g + 5, idx_c, sem_ic)

    gather_wait(idx_a, rows_a, sem_a)
    gather_wait(idx_b, rows_b, sem_b)
    idx_wait(GROUPS + 2, idx_c, sem_ic)

    plsc.subcore_barrier()
    pltpu.sync_copy(acc_sh.at[pl.ds(s * RPT, RPT)],
                    out_hbm.at[c, pl.ds(s * RPT, RPT)])


def _tcb_body(x_ref, wa_ref, wp_ref, bp_ref, w1_ref, d0_ref, d1_ref,
              h_ref, hs1_ref, dinv_ref):
    wa = wa_ref[...]
    xs = [x_ref[:, pl.ds(l * D, D)] for l in range(L)]
    sc = [jnp.sum(xl * wa, axis=1, keepdims=True) for xl in xs]
    m = sc[0]
    for l in range(1, L):
        m = jnp.maximum(m, sc[l])
    es = [jnp.exp(s - m) for s in sc]
    den = es[0]
    for l in range(1, L):
        den = den + es[l]
    pooled = (es[0] / den) * xs[0]
    for l in range(1, L):
        pooled = pooled + (es[l] / den) * xs[l]
    h = jnp.dot(pooled, wp_ref[...], preferred_element_type=jnp.float32) \
        + bp_ref[...]
    deg = d0_ref[...] + d1_ref[...] + 1.0
    dinv = lax.rsqrt(deg)
    h_ref[...] = h
    dinv_ref[...] = dinv
    hs1_ref[...] = jnp.dot(h, w1_ref[...],
                           preferred_element_type=jnp.float32) * dinv


def _tc_pool(x2d, w_att2d, W_proj, b_proj2d, W1, d0, d1):
    return pl.pallas_call(
        _tcb_body,
        grid=(GRID,),
        in_specs=[
            pl.BlockSpec((BLK, L * D), lambda i: (i, 0)),
            pl.BlockSpec((1, D), lambda i: (0, 0)),
            pl.BlockSpec((D, H), lambda i: (0, 0)),
            pl.BlockSpec((1, H), lambda i: (0, 0)),
            pl.BlockSpec((H, H), lambda i: (0, 0)),
            pl.BlockSpec((BLK, 1), lambda i: (i, 0)),
            pl.BlockSpec((BLK, 1), lambda i: (i, 0)),
        ],
        out_specs=[
            pl.BlockSpec((BLK, H), lambda i: (i, 0)),
            pl.BlockSpec((BLK, H), lambda i: (i, 0)),
            pl.BlockSpec((BLK, 1), lambda i: (i, 0)),
        ],
        out_shape=[
            jax.ShapeDtypeStruct((N, H), jnp.float32),
            jax.ShapeDtypeStruct((N, H), jnp.float32),
            jax.ShapeDtypeStruct((N, 1), jnp.float32),
        ],
    )(x2d, w_att2d, W_proj, b_proj2d, W1, d0, d1)


def _tcc_body(p0_ref, p1_ref, hs1_ref, dinv_ref, b1_ref, w2_ref, hs2_ref):
    dinv = dinv_ref[...]
    g = dinv * (p0_ref[...] + p1_ref[...] + hs1_ref[...]) + b1_ref[...]
    g = jnp.maximum(g, 0.0)
    hs2_ref[...] = jnp.dot(g, w2_ref[...],
                           preferred_element_type=jnp.float32) * dinv


def _tc_mid(p0, p1, hs1, dinv, b1_2d, W2):
    return pl.pallas_call(
        _tcc_body,
        grid=(GRID,),
        in_specs=[
            pl.BlockSpec((BLK, H), lambda i: (i, 0)),
            pl.BlockSpec((BLK, H), lambda i: (i, 0)),
            pl.BlockSpec((BLK, H), lambda i: (i, 0)),
            pl.BlockSpec((BLK, 1), lambda i: (i, 0)),
            pl.BlockSpec((1, H), lambda i: (0, 0)),
            pl.BlockSpec((H, H), lambda i: (0, 0)),
        ],
        out_specs=[pl.BlockSpec((BLK, H), lambda i: (i, 0))],
        out_shape=[jax.ShapeDtypeStruct((N, H), jnp.float32)],
    )(p0, p1, hs1, dinv, b1_2d, W2)[0]


def _tcd_body(q0_ref, q1_ref, hs2_ref, dinv_ref, b2_ref, h_ref,
              wc_ref, bc_ref, wo_ref, bo_ref, out_ref):
    o2 = dinv_ref[...] * (q0_ref[...] + q1_ref[...] + hs2_ref[...]) \
        + b2_ref[...]
    hh = o2 + h_ref[...]
    ctr = jnp.dot(hh, wc_ref[...], preferred_element_type=jnp.float32) \
        + bc_ref[...]
    z = jnp.dot(hh, wo_ref[...], preferred_element_type=jnp.float32) \
        + bo_ref[...]
    off = jnp.maximum(z, 0.0) + jnp.log1p(jnp.exp(-jnp.abs(z)))
    out_ref[:, pl.ds(0, H)] = ctr - off
    out_ref[:, pl.ds(H, H)] = ctr + off


def _tc_box(q0, q1, hs2, dinv, b2_2d, h, Wc, bc_2d, Wo, bo_2d):
    return pl.pallas_call(
        _tcd_body,
        grid=(GRID,),
        in_specs=[
            pl.BlockSpec((BLK, H), lambda i: (i, 0)),
            pl.BlockSpec((BLK, H), lambda i: (i, 0)),
            pl.BlockSpec((BLK, H), lambda i: (i, 0)),
            pl.BlockSpec((BLK, 1), lambda i: (i, 0)),
            pl.BlockSpec((1, H), lambda i: (0, 0)),
            pl.BlockSpec((BLK, H), lambda i: (i, 0)),
            pl.BlockSpec((H, H), lambda i: (0, 0)),
            pl.BlockSpec((1, H), lambda i: (0, 0)),
            pl.BlockSpec((H, H), lambda i: (0, 0)),
            pl.BlockSpec((1, H), lambda i: (0, 0)),
        ],
        out_specs=[pl.BlockSpec((BLK, 2 * H), lambda i: (i, 0))],
        out_shape=[jax.ShapeDtypeStruct((N, 2 * H), jnp.float32)],
    )(q0, q1, hs2, dinv, b2_2d, h, Wc, bc_2d, Wo, bo_2d)[0]


def kernel(x, edge_index, edge_weight, w_att, W_proj, b_proj,
           W1, b1, W2, b2, Wc, bc, Wo, bo):
    x2d = x.reshape(N, L * D)
    pad = EP - E
    src = jnp.concatenate([edge_index[0], jnp.zeros((pad,), jnp.int32)])
    dst = jnp.concatenate([edge_index[1], jnp.zeros((pad,), jnp.int32)])
    ew = jnp.concatenate([edge_weight, jnp.zeros((pad,), jnp.float32)])
    ewb = lax.bitcast_convert_type(ew, jnp.int32)
    packed = jnp.stack([src.reshape(NW, GA, GW), dst.reshape(NW, GA, GW),
                        ewb.reshape(NW, GA, GW)], axis=2)
    packed = jnp.pad(packed, ((0, 0), (0, 0), (0, 0), (0, 128 - GW)))
    dstg = dst.reshape(NW, GA, GW)
    ewg = ew.reshape(NW, GA, GW)
    zeros_deg = jnp.zeros((DPT,), jnp.float32)
    zeros_rows = jnp.zeros((RPT, H), jnp.float32)

    deg_parts = _sc_degree(dstg, ewg, zeros_deg)
    d0 = deg_parts[0, 0, :N].reshape(N, 1)
    d1 = deg_parts[1, 0, :N].reshape(N, 1)

    h, hs1, dinv = _tc_pool(x2d, w_att.reshape(1, D), W_proj,
                            b_proj.reshape(1, H), W1, d0, d1)

    m1 = _sc_message(hs1, packed, zeros_rows)
    hs2 = _tc_mid(m1[0], m1[1], hs1, dinv, b1.reshape(1, H), W2)

    m2 = _sc_message(hs2, packed, zeros_rows)
    return _tc_box(m2[0], m2[1], hs2, dinv, b2.reshape(1, H), h,
                   Wc, bc.reshape(1, H), Wo, bo.reshape(1, H))


# speedup vs baseline: 1.2156x; 1.2156x over previous
import dataclasses
import functools

import jax
import jax.numpy as jnp
from jax import lax
from jax.experimental import pallas as pl
from jax.experimental.pallas import tpu as pltpu
from jax.experimental.pallas import tpu_sc as plsc

N = 10000
E = 320000
D = 768
L = 4
H = 128
NC = 2
NS = 16
NW = NC * NS
GW = 128
GH = GW // 2
GROUPS = 80
GA = GROUPS + 2
EP = NW * GA * GW
NP = 10240
RPT = NP // NS
DPT = NP // NS

BLK = 1000
GRID = N // BLK

_mesh = plsc.VectorSubcoreMesh(core_axis_name="core", subcore_axis_name="subcore")

_sc_params = pltpu.CompilerParams()
if "needs_layout_passes" in pltpu.CompilerParams.__dataclass_fields__:
    _sc_params = dataclasses.replace(_sc_params, needs_layout_passes=False)


@functools.partial(
    pl.kernel,
    out_type=jax.ShapeDtypeStruct((NC, 1, NP), jnp.float32),
    mesh=_mesh,
    scratch_types=[
        pltpu.VMEM((GA, GW), jnp.int32),
        pltpu.VMEM((GA, GW), jnp.float32),
        pltpu.VMEM_SHARED((NP,), jnp.float32),
    ],
)
def _sc_degree(dstg_hbm, ewg_hbm, zeros_hbm, out_hbm, dst_v, ew_v, acc_sh):
    c = lax.axis_index("core")
    s = lax.axis_index("subcore")
    wid = s * NC + c
    pltpu.sync_copy(zeros_hbm, acc_sh.at[pl.ds(s * DPT, DPT)])
    plsc.subcore_barrier()
    pltpu.sync_copy(dstg_hbm.at[wid], dst_v)
    pltpu.sync_copy(ewg_hbm.at[wid], ew_v)

    @pl.loop(0, GA)
    def _(g):
        pltpu.sync_copy(ew_v.at[g], acc_sh.at[dst_v.at[g]], add=True)

    plsc.subcore_barrier()
    pltpu.sync_copy(acc_sh.at[pl.ds(s * DPT, DPT)],
                    out_hbm.at[c, 0, pl.ds(s * DPT, DPT)])


@functools.partial(
    pl.kernel,
    out_type=jax.ShapeDtypeStruct((NC, NP, H), jnp.float32),
    mesh=_mesh,
    scratch_types=[
        pltpu.VMEM((3, GW), jnp.int32),
        pltpu.VMEM((3, GW), jnp.int32),
        pltpu.VMEM((2, GH), jnp.int32),
        pltpu.VMEM((GW, H), jnp.float32),
        pltpu.VMEM((GW, H), jnp.float32),
        pltpu.VMEM((GH, H), jnp.float32),
        pltpu.VMEM_SHARED((NP, H), jnp.float32),
        pltpu.SemaphoreType.DMA,
        pltpu.SemaphoreType.DMA,
        pltpu.SemaphoreType.DMA,
        pltpu.SemaphoreType.DMA,
    ],
    compiler_params=_sc_params,
)
def _sc_message(hs_hbm, packed_hbm, zrows_hbm, out_hbm,
                idx_a, idx_b, dst2, rows_a, rows_b, rows_s, acc_sh,
                sem_ia, sem_ib, sem_a, sem_b):
    c = lax.axis_index("core")
    s = lax.axis_index("subcore")
    wid = s * NC + c
    pltpu.sync_copy(zrows_hbm, acc_sh.at[pl.ds(s * RPT, RPT)])
    plsc.subcore_barrier()

    two = lax.broadcast(2, (16,))

    def idx_dma(g, idx, sem):
        pltpu.async_copy(packed_hbm.at[wid, g], idx, sem)

    def idx_wait(g, idx, sem):
        pltpu.make_async_copy(packed_hbm.at[wid, g], idx, sem).wait()

    def gather(idx, rows, sem):
        pltpu.async_copy(hs_hbm.at[idx.at[0]], rows, sem)

    def gather_wait(idx, rows, sem):
        pltpu.make_async_copy(hs_hbm.at[idx.at[0]], rows, sem).wait()

    def process(idx, rows, sem):
        gather_wait(idx, rows, sem)
        for t in range(2):
            base = t * GH
            for k in range(GH // 16):
                dst2[t, pl.ds(k * 16, 16)] = \
                    idx[1, pl.ds(base + k * 16, 16)]

            @pl.loop(0, GH // 2)
            def _(pp):
                for u in range(2):
                    e = pp * 2 + u
                    e2 = e + base
                    wvi = plsc.load_gather(
                        idx, [two, lax.broadcast(e2, (16,))])
                    wv = plsc.bitcast(wvi, jnp.float32)
                    for k in range(H // 16):
                        rows_s[e, pl.ds(k * 16, 16)] = \
                            rows[e2, pl.ds(k * 16, 16)] * wv

            pltpu.sync_copy(rows_s, acc_sh.at[dst2.at[t]], add=True)

    idx_dma(0, idx_a, sem_ia)
    idx_wait(0, idx_a, sem_ia)
    gather(idx_a, rows_a, sem_a)
    idx_dma(1, idx_b, sem_ib)

    @pl.loop(0, GROUPS, step=2)
    def _(g):
        idx_wait(g + 1, idx_b, sem_ib)
        gather(idx_b, rows_b, sem_b)
        process(idx_a, rows_a, sem_a)
        idx_dma(g + 2, idx_a, sem_ia)
        idx_wait(g + 2, idx_a, sem_ia)
        gather(idx_a, rows_a, sem_a)
        process(idx_b, rows_b, sem_b)
        idx_dma(g + 3, idx_b, sem_ib)

    gather_wait(idx_a, rows_a, sem_a)
    idx_wait(GROUPS + 1, idx_b, sem_ib)

    plsc.subcore_barrier()
    pltpu.sync_copy(acc_sh.at[pl.ds(s * RPT, RPT)],
                    out_hbm.at[c, pl.ds(s * RPT, RPT)])


def _tcb_body(x_ref, wa_ref, wp_ref, bp_ref, w1_ref, d0_ref, d1_ref,
              h_ref, hs1_ref, dinv_ref):
    wa = wa_ref[...]
    xs = [x_ref[:, pl.ds(l * D, D)] for l in range(L)]
    sc = [jnp.sum(xl * wa, axis=1, keepdims=True) for xl in xs]
    m = sc[0]
    for l in range(1, L):
        m = jnp.maximum(m, sc[l])
    es = [jnp.exp(s - m) for s in sc]
    den = es[0]
    for l in range(1, L):
        den = den + es[l]
    pooled = (es[0] / den) * xs[0]
    for l in range(1, L):
        pooled = pooled + (es[l] / den) * xs[l]
    h = jnp.dot(pooled, wp_ref[...], preferred_element_type=jnp.float32) \
        + bp_ref[...]
    deg = d0_ref[...] + d1_ref[...] + 1.0
    dinv = lax.rsqrt(deg)
    h_ref[...] = h
    dinv_ref[...] = dinv
    hs1_ref[...] = jnp.dot(h, w1_ref[...],
                           preferred_element_type=jnp.float32) * dinv


def _tc_pool(x2d, w_att2d, W_proj, b_proj2d, W1, d0, d1):
    return pl.pallas_call(
        _tcb_body,
        grid=(GRID,),
        in_specs=[
            pl.BlockSpec((BLK, L * D), lambda i: (i, 0)),
            pl.BlockSpec((1, D), lambda i: (0, 0)),
            pl.BlockSpec((D, H), lambda i: (0, 0)),
            pl.BlockSpec((1, H), lambda i: (0, 0)),
            pl.BlockSpec((H, H), lambda i: (0, 0)),
            pl.BlockSpec((BLK, 1), lambda i: (i, 0)),
            pl.BlockSpec((BLK, 1), lambda i: (i, 0)),
        ],
        out_specs=[
            pl.BlockSpec((BLK, H), lambda i: (i, 0)),
            pl.BlockSpec((BLK, H), lambda i: (i, 0)),
            pl.BlockSpec((BLK, 1), lambda i: (i, 0)),
        ],
        out_shape=[
            jax.ShapeDtypeStruct((N, H), jnp.float32),
            jax.ShapeDtypeStruct((N, H), jnp.float32),
            jax.ShapeDtypeStruct((N, 1), jnp.float32),
        ],
    )(x2d, w_att2d, W_proj, b_proj2d, W1, d0, d1)


def _tcc_body(p0_ref, p1_ref, hs1_ref, dinv_ref, b1_ref, w2_ref, hs2_ref):
    dinv = dinv_ref[...]
    g = dinv * (p0_ref[...] + p1_ref[...] + hs1_ref[...]) + b1_ref[...]
    g = jnp.maximum(g, 0.0)
    hs2_ref[...] = jnp.dot(g, w2_ref[...],
                           preferred_element_type=jnp.float32) * dinv


def _tc_mid(p0, p1, hs1, dinv, b1_2d, W2):
    return pl.pallas_call(
        _tcc_body,
        grid=(GRID,),
        in_specs=[
            pl.BlockSpec((BLK, H), lambda i: (i, 0)),
            pl.BlockSpec((BLK, H), lambda i: (i, 0)),
            pl.BlockSpec((BLK, H), lambda i: (i, 0)),
            pl.BlockSpec((BLK, 1), lambda i: (i, 0)),
            pl.BlockSpec((1, H), lambda i: (0, 0)),
            pl.BlockSpec((H, H), lambda i: (0, 0)),
        ],
        out_specs=[pl.BlockSpec((BLK, H), lambda i: (i, 0))],
        out_shape=[jax.ShapeDtypeStruct((N, H), jnp.float32)],
    )(p0, p1, hs1, dinv, b1_2d, W2)[0]


def _tcd_body(q0_ref, q1_ref, hs2_ref, dinv_ref, b2_ref, h_ref,
              wc_ref, bc_ref, wo_ref, bo_ref, out_ref):
    o2 = dinv_ref[...] * (q0_ref[...] + q1_ref[...] + hs2_ref[...]) \
        + b2_ref[...]
    hh = o2 + h_ref[...]
    ctr = jnp.dot(hh, wc_ref[...], preferred_element_type=jnp.float32) \
        + bc_ref[...]
    z = jnp.dot(hh, wo_ref[...], preferred_element_type=jnp.float32) \
        + bo_ref[...]
    off = jnp.maximum(z, 0.0) + jnp.log1p(jnp.exp(-jnp.abs(z)))
    out_ref[:, pl.ds(0, H)] = ctr - off
    out_ref[:, pl.ds(H, H)] = ctr + off


def _tc_box(q0, q1, hs2, dinv, b2_2d, h, Wc, bc_2d, Wo, bo_2d):
    return pl.pallas_call(
        _tcd_body,
        grid=(GRID,),
        in_specs=[
            pl.BlockSpec((BLK, H), lambda i: (i, 0)),
            pl.BlockSpec((BLK, H), lambda i: (i, 0)),
            pl.BlockSpec((BLK, H), lambda i: (i, 0)),
            pl.BlockSpec((BLK, 1), lambda i: (i, 0)),
            pl.BlockSpec((1, H), lambda i: (0, 0)),
            pl.BlockSpec((BLK, H), lambda i: (i, 0)),
            pl.BlockSpec((H, H), lambda i: (0, 0)),
            pl.BlockSpec((1, H), lambda i: (0, 0)),
            pl.BlockSpec((H, H), lambda i: (0, 0)),
            pl.BlockSpec((1, H), lambda i: (0, 0)),
        ],
        out_specs=[pl.BlockSpec((BLK, 2 * H), lambda i: (i, 0))],
        out_shape=[jax.ShapeDtypeStruct((N, 2 * H), jnp.float32)],
    )(q0, q1, hs2, dinv, b2_2d, h, Wc, bc_2d, Wo, bo_2d)[0]


def kernel(x, edge_index, edge_weight, w_att, W_proj, b_proj,
           W1, b1, W2, b2, Wc, bc, Wo, bo):
    x2d = x.reshape(N, L * D)
    pad = EP - E
    src = jnp.concatenate([edge_index[0], jnp.zeros((pad,), jnp.int32)])
    dst = jnp.concatenate([edge_index[1], jnp.zeros((pad,), jnp.int32)])
    ew = jnp.concatenate([edge_weight, jnp.zeros((pad,), jnp.float32)])
    ewb = lax.bitcast_convert_type(ew, jnp.int32)
    packed = jnp.stack([src.reshape(NW, GA, GW), dst.reshape(NW, GA, GW),
                        ewb.reshape(NW, GA, GW)], axis=2)
    dstg = dst.reshape(NW, GA, GW)
    ewg = ew.reshape(NW, GA, GW)
    zeros_deg = jnp.zeros((DPT,), jnp.float32)
    zeros_rows = jnp.zeros((RPT, H), jnp.float32)

    deg_parts = _sc_degree(dstg, ewg, zeros_deg)
    d0 = deg_parts[0, 0, :N].reshape(N, 1)
    d1 = deg_parts[1, 0, :N].reshape(N, 1)

    h, hs1, dinv = _tc_pool(x2d, w_att.reshape(1, D), W_proj,
                            b_proj.reshape(1, H), W1, d0, d1)

    m1 = _sc_message(hs1, packed, zeros_rows)
    hs2 = _tc_mid(m1[0], m1[1], hs1, dinv, b1.reshape(1, H), W2)

    m2 = _sc_message(hs2, packed, zeros_rows)
    return _tc_box(m2[0], m2[1], hs2, dinv, b2.reshape(1, H), h,
                   Wc, bc.reshape(1, H), Wo, bo.reshape(1, H))
